# baseline (speedup 1.0000x reference)
import jax
import jax.numpy as jnp
from jax import lax
from jax.experimental import pallas as pl
from jax.experimental.pallas import tpu as pltpu
from jax.experimental.pallas import tpu_sc as plsc

N_CORES = 2
N_SUBCORES = 16
N_WORKERS = N_CORES * N_SUBCORES

N_ATOMS = 50000

CB = 10
BP = 128 * CB
BW = 512 * CB
NBUF = 3

HI_MASK = -65536


def _body(ta_hbm, tb_hbm, idxi_hbm, idxj_hbm, out_hbm, ta_v, tb_v,
          ii0, ii1, ii2, jj0, jj1, jj2, ob0, ob1, ob2,
          si0, si1, si2, sj0, sj1, sj2, so0, so1, so2):
    wid = lax.axis_index("s") * N_CORES + lax.axis_index("c")
    n_pairs = idxi_hbm.shape[0]
    nblocks = n_pairs // BP
    nm = (nblocks + N_WORKERS - 1) // N_WORKERS

    ii = (ii0, ii1, ii2)
    jj = (jj0, jj1, jj2)
    ob = (ob0, ob1, ob2)
    si = (si0, si1, si2)
    sj = (sj0, sj1, sj2)
    so = (so0, so1, so2)

    def start_in(m, ph):
        blk = wid + N_WORKERS * m

        @pl.when(blk < nblocks)
        def _():
            base = BP * blk
            pltpu.async_copy(idxi_hbm.at[pl.ds(base, BP)], ii[ph], si[ph])
            pltpu.async_copy(idxj_hbm.at[pl.ds(base, BP)], jj[ph], sj[ph])

    def compute(ii_v, jj_v, ob_v):
        @pl.loop(0, BP // 16, unroll=8)
        def _vec(t):
            base = 16 * t
            vj = jj_v[pl.ds(base, 16)]
            vi = ii_v[pl.ds(base, 16)]
            aj = plsc.load_gather(ta_v, [vj])
            ai = plsc.load_gather(ta_v, [vi])
            bj = plsc.load_gather(tb_v, [vj])
            bi = plsc.load_gather(tb_v, [vi])
            dxy = plsc.bitcast(plsc.bitcast(aj, jnp.bfloat16)
                               - plsc.bitcast(ai, jnp.bfloat16), jnp.int32)
            dx = plsc.bitcast(dxy & HI_MASK, jnp.float32)
            dy = plsc.bitcast(dxy << 16, jnp.float32)
            dz = (plsc.bitcast(bj, jnp.float32)
                  - plsc.bitcast(bi, jnp.float32))
            u = base // 128
            lo = base % 128
            ob_v[pl.ds(512 * u + lo, 16)] = dx
            ob_v[pl.ds(512 * u + 128 + lo, 16)] = dy
            ob_v[pl.ds(512 * u + 256 + lo, 16)] = dz

    def step(m, ph):
        blk = wid + N_WORKERS * m
        start_in(m + NBUF - 1, (ph + NBUF - 1) % NBUF)

        @pl.when(blk < nblocks)
        def _():
            @pl.when(m >= NBUF)
            def _():
                pltpu.make_async_copy(
                    ob[ph], out_hbm.at[pl.ds(0, BW)], so[ph]).wait()
            pltpu.make_async_copy(idxi_hbm.at[pl.ds(0, BP)], ii[ph], si[ph]).wait()
            pltpu.make_async_copy(idxj_hbm.at[pl.ds(0, BP)], jj[ph], sj[ph]).wait()
            compute(ii[ph], jj[ph], ob[ph])
            pltpu.async_copy(ob[ph], out_hbm.at[pl.ds(BW * blk, BW)], so[ph])

    for m0 in range(NBUF - 1):
        start_in(m0, m0 % NBUF)
    pltpu.sync_copy(ta_hbm, ta_v)
    pltpu.sync_copy(tb_hbm, tb_v)

    nh = (nm + NBUF - 1) // NBUF

    @pl.loop(0, nh)
    def _h(h):
        for k in range(NBUF):
            step(NBUF * h + k, k)

    for m in range(NBUF * nh - NBUF, NBUF * nh):
        blk = wid + N_WORKERS * m

        @pl.when(blk < nblocks)
        def _():
            pltpu.make_async_copy(
                ob[m % NBUF], out_hbm.at[pl.ds(0, BW)], so[m % NBUF]).wait()


def kernel(R, offsets, idx_i, idx_j):
    n_pairs = idx_i.shape[0]
    n_chunks = n_pairs // 128
    u = lax.bitcast_convert_type(R.astype(jnp.bfloat16), jnp.uint16)
    x = u[:, 0].astype(jnp.uint32)
    y = u[:, 1].astype(jnp.uint32)
    z = u[:, 2]
    ta = ((x << 16) | y).astype(jnp.int32)
    tb = (z.astype(jnp.uint32) << 16).astype(jnp.int32)
    idx_i = idx_i.astype(jnp.int32)
    idx_j = idx_j.astype(jnp.int32)

    mesh = plsc.VectorSubcoreMesh(core_axis_name="c", subcore_axis_name="s",
                                  num_cores=N_CORES, num_subcores=N_SUBCORES)
    run = pl.kernel(
        _body,
        out_type=jax.ShapeDtypeStruct((n_chunks * 512,), jnp.float32),
        mesh=mesh,
        scratch_types=(
            [pltpu.VMEM((N_ATOMS,), jnp.int32),
             pltpu.VMEM((N_ATOMS,), jnp.int32)]
            + [pltpu.VMEM((BP,), jnp.int32)] * 6
            + [pltpu.VMEM((BW,), jnp.float32)] * 3
            + [pltpu.SemaphoreType.DMA] * 9
        ),
        compiler_params=pltpu.CompilerParams(needs_layout_passes=False),
    )
    flat = run(ta, tb, idx_i, idx_j)
    d = jnp.swapaxes(flat.reshape(n_chunks, 4, 128), 1, 2).reshape(n_pairs, 4)[:, :3]
    return d + offsets


# speedup vs baseline: 1.0044x; 1.0044x over previous
import jax
import jax.numpy as jnp
from jax import lax
from jax.experimental import pallas as pl
from jax.experimental.pallas import tpu as pltpu
from jax.experimental.pallas import tpu_sc as plsc

N_CORES = 2
N_SUBCORES = 16
N_WORKERS = N_CORES * N_SUBCORES

N_ATOMS = 50000

CB = 20
BP = 128 * CB
BW = 512 * CB
NBUF = 2

HI_MASK = -65536


def _body(ta_hbm, tb_hbm, idxi_hbm, idxj_hbm, out_hbm, ta_v, tb_v,
          ii0, ii1, jj0, jj1, ob0, ob1,
          si0, si1, sj0, sj1, so0, so1):
    wid = lax.axis_index("s") * N_CORES + lax.axis_index("c")
    n_pairs = idxi_hbm.shape[0]
    nblocks = n_pairs // BP
    nm = (nblocks + N_WORKERS - 1) // N_WORKERS

    ii = (ii0, ii1)
    jj = (jj0, jj1)
    ob = (ob0, ob1)
    si = (si0, si1)
    sj = (sj0, sj1)
    so = (so0, so1)

    def start_in(m, ph):
        blk = wid + N_WORKERS * m

        @pl.when(blk < nblocks)
        def _():
            base = BP * blk
            pltpu.async_copy(idxi_hbm.at[pl.ds(base, BP)], ii[ph], si[ph])
            pltpu.async_copy(idxj_hbm.at[pl.ds(base, BP)], jj[ph], sj[ph])

    def compute(ii_v, jj_v, ob_v):
        @pl.loop(0, BP // 16, unroll=8)
        def _vec(t):
            base = 16 * t
            vj = jj_v[pl.ds(base, 16)]
            vi = ii_v[pl.ds(base, 16)]
            aj = plsc.load_gather(ta_v, [vj])
            ai = plsc.load_gather(ta_v, [vi])
            bj = plsc.load_gather(tb_v, [vj])
            bi = plsc.load_gather(tb_v, [vi])
            dxy = plsc.bitcast(plsc.bitcast(aj, jnp.bfloat16)
                               - plsc.bitcast(ai, jnp.bfloat16), jnp.int32)
            dx = plsc.bitcast(dxy & HI_MASK, jnp.float32)
            dy = plsc.bitcast(dxy << 16, jnp.float32)
            dz = (plsc.bitcast(bj, jnp.float32)
                  - plsc.bitcast(bi, jnp.float32))
            u = base // 128
            lo = base % 128
            ob_v[pl.ds(512 * u + lo, 16)] = dx
            ob_v[pl.ds(512 * u + 128 + lo, 16)] = dy
            ob_v[pl.ds(512 * u + 256 + lo, 16)] = dz

    def step(m, ph):
        blk = wid + N_WORKERS * m
        start_in(m + NBUF - 1, (ph + NBUF - 1) % NBUF)

        @pl.when(blk < nblocks)
        def _():
            @pl.when(m >= NBUF)
            def _():
                pltpu.make_async_copy(
                    ob[ph], out_hbm.at[pl.ds(0, BW)], so[ph]).wait()
            pltpu.make_async_copy(idxi_hbm.at[pl.ds(0, BP)], ii[ph], si[ph]).wait()
            pltpu.make_async_copy(idxj_hbm.at[pl.ds(0, BP)], jj[ph], sj[ph]).wait()
            compute(ii[ph], jj[ph], ob[ph])
            pltpu.async_copy(ob[ph], out_hbm.at[pl.ds(BW * blk, BW)], so[ph])

    for m0 in range(NBUF - 1):
        start_in(m0, m0 % NBUF)
    pltpu.sync_copy(ta_hbm, ta_v)
    pltpu.sync_copy(tb_hbm, tb_v)

    nh = (nm + NBUF - 1) // NBUF

    @pl.loop(0, nh)
    def _h(h):
        for k in range(NBUF):
            step(NBUF * h + k, k)

    for m in range(NBUF * nh - NBUF, NBUF * nh):
        blk = wid + N_WORKERS * m

        @pl.when(blk < nblocks)
        def _():
            pltpu.make_async_copy(
                ob[m % NBUF], out_hbm.at[pl.ds(0, BW)], so[m % NBUF]).wait()


def kernel(R, offsets, idx_i, idx_j):
    n_pairs = idx_i.shape[0]
    n_chunks = n_pairs // 128
    u = lax.bitcast_convert_type(R.astype(jnp.bfloat16), jnp.uint16)
    x = u[:, 0].astype(jnp.uint32)
    y = u[:, 1].astype(jnp.uint32)
    z = u[:, 2]
    ta = ((x << 16) | y).astype(jnp.int32)
    tb = (z.astype(jnp.uint32) << 16).astype(jnp.int32)
    idx_i = idx_i.astype(jnp.int32)
    idx_j = idx_j.astype(jnp.int32)

    mesh = plsc.VectorSubcoreMesh(core_axis_name="c", subcore_axis_name="s",
                                  num_cores=N_CORES, num_subcores=N_SUBCORES)
    run = pl.kernel(
        _body,
        out_type=jax.ShapeDtypeStruct((n_chunks * 512,), jnp.float32),
        mesh=mesh,
        scratch_types=(
            [pltpu.VMEM((N_ATOMS,), jnp.int32),
             pltpu.VMEM((N_ATOMS,), jnp.int32)]
            + [pltpu.VMEM((BP,), jnp.int32)] * 4
            + [pltpu.VMEM((BW,), jnp.float32)] * 2
            + [pltpu.SemaphoreType.DMA] * 6
        ),
        compiler_params=pltpu.CompilerParams(needs_layout_passes=False),
    )
    flat = run(ta, tb, idx_i, idx_j)
    d = jnp.swapaxes(flat.reshape(n_chunks, 4, 128), 1, 2).reshape(n_pairs, 4)[:, :3]
    return d + offsets
